# Initial kernel scaffold; baseline (speedup 1.0000x reference)
import jax
import jax.numpy as jnp
from jax.experimental import pallas as pl


def kernel(IX, weight):
    raise NotImplementedError("write your pallas kernel here")


# speedup vs baseline: 1.4074x; 1.4074x over previous
import functools

import jax
import jax.numpy as jnp
from jax import lax
from jax.experimental import pallas as pl
from jax.experimental.pallas import tpu as pltpu
from jax.experimental.pallas import tpu_sc as plsc

D = 32
NW = 32
CHUNK = 128


@functools.partial(jax.jit, static_argnums=(2, 3))
def _emb_lookup(ix, weight, per_w, nchunk):
    mesh = plsc.VectorSubcoreMesh(core_axis_name="c", subcore_axis_name="s")
    n = NW * per_w

    @functools.partial(
        pl.kernel,
        mesh=mesh,
        out_type=jax.ShapeDtypeStruct((n, D), jnp.float32),
        scratch_types=[
            pltpu.VMEM((nchunk, CHUNK), jnp.int32),
            pltpu.VMEM((CHUNK, D), jnp.float32),
            pltpu.SemaphoreType.DMA,
        ],
        compiler_params=pltpu.CompilerParams(use_tc_tiling_on_sc=False),
    )
    def emb_kernel(ix_hbm, w_hbm, out_hbm, idx_v, buf, sem):
        wid = lax.axis_index("s") * 2 + lax.axis_index("c")
        base = wid * per_w
        pltpu.sync_copy(ix_hbm.at[wid], idx_v)

        def body(j, carry):
            pltpu.async_copy(w_hbm.at[idx_v.at[j]], buf, sem).wait()
            pltpu.sync_copy(buf, out_hbm.at[pl.ds(base + j * CHUNK, CHUNK)])
            return carry

        lax.fori_loop(0, nchunk, body, 0)

    return emb_kernel(ix, weight)


def kernel(IX, weight):
    b, t = IX.shape
    n = b * t
    per_w = n // NW
    nchunk = per_w // CHUNK
    ix = IX.reshape(NW, nchunk, CHUNK).astype(jnp.int32)
    out = _emb_lookup(ix, weight, per_w, nchunk)
    return out.reshape(b, t, D)


# speedup vs baseline: 1.5116x; 1.0740x over previous
import functools

import jax
import jax.numpy as jnp
from jax import lax
from jax.experimental import pallas as pl
from jax.experimental.pallas import tpu as pltpu
from jax.experimental.pallas import tpu_sc as plsc

D = 32
NW = 32
CHUNK = 128


G = 8


@functools.partial(jax.jit, static_argnums=(2, 3))
def _emb_lookup(ix, weight, per_w, nchunk):
    mesh = plsc.VectorSubcoreMesh(core_axis_name="c", subcore_axis_name="s")
    n = NW * per_w
    ngroups = nchunk // G
    npairs = ngroups // 2
    gc = G * CHUNK

    @functools.partial(
        pl.kernel,
        mesh=mesh,
        out_type=jax.ShapeDtypeStruct((n, D), jnp.float32),
        scratch_types=[
            pltpu.VMEM((nchunk, CHUNK), jnp.int32),
            pltpu.VMEM((gc, D), jnp.float32),
            pltpu.VMEM((gc, D), jnp.float32),
            pltpu.SemaphoreType.DMA,
            pltpu.SemaphoreType.DMA,
            pltpu.SemaphoreType.DMA,
            pltpu.SemaphoreType.DMA,
        ],
        compiler_params=pltpu.CompilerParams(use_tc_tiling_on_sc=False),
    )
    def emb_kernel(ix_hbm, w_hbm, out_hbm, idx_v, buf0, buf1, gs0, gs1, os0, os1):
        wid = lax.axis_index("s") * 2 + lax.axis_index("c")
        base = wid * per_w
        pltpu.sync_copy(ix_hbm.at[wid], idx_v)

        def fire(goff, buf, gsem):
            for b in range(G):
                pltpu.async_copy(
                    w_hbm.at[idx_v.at[goff * G + b]],
                    buf.at[pl.ds(b * CHUNK, CHUNK)],
                    gsem,
                )

        def drain_gathers(buf, gsem):
            pltpu.make_async_copy(w_hbm.at[pl.ds(0, gc)], buf, gsem).wait()

        def start_out(goff, buf, osem):
            pltpu.async_copy(buf, out_hbm.at[pl.ds(base + goff * gc, gc)], osem)

        def wait_out(buf, osem):
            pltpu.make_async_copy(buf, out_hbm.at[pl.ds(base, gc)], osem).wait()

        def body(p, carry):
            for buf, gsem, osem, off in ((buf0, gs0, os0, 0), (buf1, gs1, os1, 1)):
                goff = 2 * p + off

                @pl.when(p > 0)
                def _():
                    wait_out(buf, osem)

                fire(goff, buf, gsem)
            for buf, gsem, osem, off in ((buf0, gs0, os0, 0), (buf1, gs1, os1, 1)):
                goff = 2 * p + off
                drain_gathers(buf, gsem)
                start_out(goff, buf, osem)
            return carry

        lax.fori_loop(0, npairs, body, 0)
        wait_out(buf0, os0)
        wait_out(buf1, os1)

    return emb_kernel(ix, weight)


def kernel(IX, weight):
    b, t = IX.shape
    n = b * t
    per_w = n // NW
    nchunk = per_w // CHUNK
    ix = IX.reshape(NW, nchunk, CHUNK).astype(jnp.int32)
    out = _emb_lookup(ix, weight, per_w, nchunk)
    return out.reshape(b, t, D)
